# Initial kernel scaffold; baseline (speedup 1.0000x reference)
import jax
import jax.numpy as jnp
from jax.experimental import pallas as pl


def kernel(x, edge_index, W1_1, b1_1, g1, bt1, W1_2, b1_2, W2_1, b2_1, g2, bt2, W2_2, b2_2, W3_1, b3_1, g3, bt3, W3_2, b3_2):
    raise NotImplementedError("write your pallas kernel here")


# speedup vs baseline: 1.6971x; 1.6971x over previous
import functools

import jax
import jax.numpy as jnp
from jax import lax
from jax.experimental import pallas as pl
from jax.experimental.pallas import tpu as pltpu
from jax.experimental.pallas import tpu_sc as plsc

N = 50000
E = 800000
H = 128
NP = 50176
BR = 512
NG = NP // BR

NW = 32
NBUCK = 64
BWN = 800
CAP = 20000
ECH = 4000
K = 64
SINK = 800
ZR = 89

_mesh = lambda: plsc.VectorSubcoreMesh(core_axis_name="c", subcore_axis_name="s")
_sc_params = pltpu.CompilerParams(needs_layout_passes=False)


def _wid():
    return lax.axis_index("s") * 2 + lax.axis_index("c")


@functools.partial(
    pl.kernel,
    out_type=(
        jax.ShapeDtypeStruct((NBUCK * CAP,), jnp.int32),
        jax.ShapeDtypeStruct((NBUCK * CAP,), jnp.int32),
        jax.ShapeDtypeStruct((NBUCK * 16,), jnp.int32),
    ),
    mesh=_mesh(),
    compiler_params=_sc_params,
    scratch_types=[
        pltpu.VMEM((ECH,), jnp.int32),
        pltpu.VMEM((ECH,), jnp.int32),
        pltpu.VMEM((CAP,), jnp.int32),
        pltpu.VMEM((CAP,), jnp.int32),
        pltpu.VMEM((CAP,), jnp.int32),
        pltpu.VMEM((CAP,), jnp.int32),
        pltpu.VMEM((16,), jnp.int32),
    ],
)
def _partition(src_hbm, dst_hbm, psrc, pdloc, pcnt,
               sbuf, dbuf, os0, od0, os1, od1, cntbuf):
    w = _wid()
    lo0 = w * BWN
    lo1 = (w + NW) * BWN

    def chunk(c, carry):
        o0, o1 = carry
        pltpu.sync_copy(src_hbm.at[pl.ds(c * ECH, ECH)], sbuf)
        pltpu.sync_copy(dst_hbm.at[pl.ds(c * ECH, ECH)], dbuf)

        def vec(i, carry):
            o0, o1 = carry
            d = dbuf[pl.ds(i * 16, 16)]
            s = sbuf[pl.ds(i * 16, 16)]
            m0 = (d >= lo0) & (d < lo0 + BWN)
            m1 = (d >= lo1) & (d < lo1 + BWN)
            pc0 = plsc.cumsum(m0.astype(jnp.int32))
            pc1 = plsc.cumsum(m1.astype(jnp.int32))
            i0 = o0 + pc0 - 1
            i1 = o1 + pc1 - 1
            plsc.store_scatter(od0, [i0], d - lo0, mask=m0)
            plsc.store_scatter(os0, [i0], s, mask=m0)
            plsc.store_scatter(od1, [i1], d - lo1, mask=m1)
            plsc.store_scatter(os1, [i1], s, mask=m1)
            return o0 + pc0[15], o1 + pc1[15]

        return lax.fori_loop(0, ECH // 16, vec, (o0, o1))

    o0, o1 = lax.fori_loop(0, E // ECH, chunk, (jnp.int32(0), jnp.int32(0)))

    sink_d = jnp.full((16,), SINK, jnp.int32)
    sink_s = jnp.zeros((16,), jnp.int32)
    for j, (osb, odb, o) in enumerate(((os0, od0, o0), (os1, od1, o1))):
        b = w + NW * j
        for t in range(4):
            odb[pl.ds(o + t * 16, 16)] = sink_d
            osb[pl.ds(o + t * 16, 16)] = sink_s
        cnt_pad = ((o + K - 1) // K) * K
        cntbuf[...] = jnp.full((16,), 0, jnp.int32) + cnt_pad
        pltpu.sync_copy(osb, psrc.at[pl.ds(b * CAP, CAP)])
        pltpu.sync_copy(odb, pdloc.at[pl.ds(b * CAP, CAP)])
        pltpu.sync_copy(cntbuf, pcnt.at[pl.ds(b * 16, 16)])


@functools.partial(
    pl.kernel,
    out_type=jax.ShapeDtypeStruct((NP, H), jnp.float32),
    mesh=_mesh(),
    compiler_params=_sc_params,
    scratch_types=[
        pltpu.VMEM((2, K), jnp.int32),
        pltpu.VMEM((K, H), jnp.float32),
        pltpu.VMEM((SINK + 1, H), jnp.float32),
        pltpu.VMEM((16,), jnp.int32),
        pltpu.SemaphoreType.DMA,
    ],
)
def _agg(h_hbm, psrc, pdloc, pcnt, out_hbm, idxb, rows, acc, cntv, sem):
    w = _wid()
    zero = jnp.zeros((16,), jnp.float32)
    iota = lax.iota(jnp.int32, 16)

    for j in range(2):
        b = w + NW * j

        def zrow(r, _):
            for f in range(H // 16):
                acc[r, pl.ds(f * 16, 16)] = zero
            return 0

        lax.fori_loop(0, SINK + 1, zrow, 0)

        pltpu.sync_copy(pcnt.at[pl.ds(b * 16, 16)], cntv)
        trips = jnp.max(cntv[...]) // K

        def edge_chunk(t, _):
            pltpu.sync_copy(psrc.at[pl.ds(b * CAP + t * K, K)], idxb.at[0])
            pltpu.sync_copy(pdloc.at[pl.ds(b * CAP + t * K, K)], idxb.at[1])
            pltpu.async_copy(h_hbm.at[idxb.at[0]], rows, sem).wait()

            def edge(e, _):
                dsp = plsc.load_gather(idxb.at[1], [jnp.full((16,), e, jnp.int32)])
                for f in range(H // 16):
                    val = rows[e, pl.ds(f * 16, 16)]
                    plsc.addupdate_scatter(acc, [dsp, f * 16 + iota], val)
                return 0

            lax.fori_loop(0, K, edge, 0)
            return 0

        lax.fori_loop(0, trips, edge_chunk, 0)

        @pl.when(b < NBUCK - 2)
        def _():
            pltpu.sync_copy(acc.at[pl.ds(0, BWN)],
                            out_hbm.at[pl.ds(b * BWN, BWN)])

        @pl.when(b == NBUCK - 2)
        def _():
            pltpu.sync_copy(acc.at[pl.ds(0, N - (NBUCK - 2) * BWN)],
                            out_hbm.at[pl.ds((NBUCK - 2) * BWN,
                                             N - (NBUCK - 2) * BWN)])


def _pre_body(xin, wa, y_ref):
    y_ref[...] = jnp.dot(xin[...], wa[...], preferred_element_type=jnp.float32)


def _pre(xin, wa):
    return pl.pallas_call(
        _pre_body,
        grid=(NG,),
        in_specs=[
            pl.BlockSpec((BR, 16), lambda i: (i, 0)),
            pl.BlockSpec((16, H), lambda i: (0, 0)),
        ],
        out_specs=pl.BlockSpec((BR, H), lambda i: (i, 0)),
        out_shape=jax.ShapeDtypeStruct((NP, H), jnp.float32),
    )(xin, wa)


def _stats_tail(i, u, u_ref, sums_ref):
    rows = i * BR + lax.broadcasted_iota(jnp.int32, (BR, 1), 0)
    u = jnp.where(rows < N, u, 0.0)
    u_ref[...] = u

    @pl.when(i == 0)
    def _():
        sums_ref[...] = jnp.zeros_like(sums_ref)

    sums_ref[0:1, :] += jnp.sum(u, axis=0, keepdims=True)
    sums_ref[1:2, :] += jnp.sum(u * u, axis=0, keepdims=True)


def _ka_body(xin, aggr, wa, ba, u_ref, sums_ref):
    i = pl.program_id(0)
    xa = xin[...] + aggr[...]
    u = jnp.dot(xa, wa[...], preferred_element_type=jnp.float32) + ba[...]
    _stats_tail(i, u, u_ref, sums_ref)


def _ka_add_body(y0, aggr, ba, u_ref, sums_ref):
    i = pl.program_id(0)
    u = y0[...] + aggr[...] + ba[...]
    _stats_tail(i, u, u_ref, sums_ref)


_KA_OUT = [
    jax.ShapeDtypeStruct((NP, H), jnp.float32),
    jax.ShapeDtypeStruct((2, H), jnp.float32),
]
_KA_OUT_SPECS = [
    pl.BlockSpec((BR, H), lambda i: (i, 0)),
    pl.BlockSpec((2, H), lambda i: (0, 0)),
]


def _ka(xin, aggr, wa, ba):
    return pl.pallas_call(
        _ka_body,
        grid=(NG,),
        in_specs=[
            pl.BlockSpec((BR, H), lambda i: (i, 0)),
            pl.BlockSpec((BR, H), lambda i: (i, 0)),
            pl.BlockSpec((H, H), lambda i: (0, 0)),
            pl.BlockSpec((1, H), lambda i: (0, 0)),
        ],
        out_specs=_KA_OUT_SPECS,
        out_shape=_KA_OUT,
    )(xin, aggr, wa, ba.reshape(1, H))


def _ka_add(y0, aggr, ba):
    return pl.pallas_call(
        _ka_add_body,
        grid=(NG,),
        in_specs=[
            pl.BlockSpec((BR, H), lambda i: (i, 0)),
            pl.BlockSpec((BR, H), lambda i: (i, 0)),
            pl.BlockSpec((1, H), lambda i: (0, 0)),
        ],
        out_specs=_KA_OUT_SPECS,
        out_shape=_KA_OUT,
    )(y0, aggr, ba.reshape(1, H))


def _bn_relu(u, sums, g, bt):
    m = sums[0:1, :] * (1.0 / N)
    var = sums[1:2, :] * (1.0 / N) - m * m
    inv = g[...] * lax.rsqrt(var + 1e-5)
    return jnp.maximum(u[...] * inv + (bt[...] - m * inv), 0.0)


def _kb_body(u, sums, g, bt, wb, bb, h_ref):
    t = _bn_relu(u, sums, g, bt)
    h_ref[...] = jnp.maximum(
        jnp.dot(t, wb[...], preferred_element_type=jnp.float32) + bb[...], 0.0)


def _kb3_body(u, sums, g, bt, wb, bb, out_ref):
    i = pl.program_id(0)
    t = _bn_relu(u, sums, g, bt)
    h = jnp.maximum(
        jnp.dot(t, wb[...], preferred_element_type=jnp.float32) + bb[...], 0.0)
    rows = i * BR + lax.broadcasted_iota(jnp.int32, (BR, 1), 0)
    h = jnp.where(rows < N, h, 0.0)

    @pl.when(i == 0)
    def _():
        out_ref[...] = jnp.zeros_like(out_ref)

    out_ref[...] += jnp.sum(h, axis=0, keepdims=True)


def _kb(u, sums, g, bt, wb, bb, last):
    in_specs = [
        pl.BlockSpec((BR, H), lambda i: (i, 0)),
        pl.BlockSpec((2, H), lambda i: (0, 0)),
        pl.BlockSpec((1, H), lambda i: (0, 0)),
        pl.BlockSpec((1, H), lambda i: (0, 0)),
        pl.BlockSpec((H, H), lambda i: (0, 0)),
        pl.BlockSpec((1, H), lambda i: (0, 0)),
    ]
    if last:
        out_spec = pl.BlockSpec((1, H), lambda i: (0, 0))
        out_shape = jax.ShapeDtypeStruct((1, H), jnp.float32)
        body = _kb3_body
    else:
        out_spec = pl.BlockSpec((BR, H), lambda i: (i, 0))
        out_shape = jax.ShapeDtypeStruct((NP, H), jnp.float32)
        body = _kb_body
    return pl.pallas_call(
        body, grid=(NG,), in_specs=in_specs, out_specs=out_spec,
        out_shape=out_shape,
    )(u, sums, g.reshape(1, H), bt.reshape(1, H), wb, bb.reshape(1, H))


def kernel(x, edge_index,
           W1_1, b1_1, g1, bt1, W1_2, b1_2,
           W2_1, b2_1, g2, bt2, W2_2, b2_2,
           W3_1, b3_1, g3, bt3, W3_2, b3_2):
    src = edge_index[0]
    dst = edge_index[1]

    x_p = jnp.zeros((NP, 16), jnp.float32).at[:N, :9].set(x)
    w1_p = jnp.zeros((16, H), jnp.float32).at[:9, :].set(W1_1)

    psrc, pdloc, pcnt = _partition(src, dst)

    y0 = _pre(x_p, w1_p)
    agg1 = _agg(y0, psrc, pdloc, pcnt)
    u1, s1 = _ka_add(y0, agg1, b1_1)
    h1 = _kb(u1, s1, g1, bt1, W1_2, b1_2, last=False)

    agg2 = _agg(h1, psrc, pdloc, pcnt)
    u2, s2 = _ka(h1, agg2, W2_1, b2_1)
    h2 = _kb(u2, s2, g2, bt2, W2_2, b2_2, last=False)

    agg3 = _agg(h2, psrc, pdloc, pcnt)
    u3, s3 = _ka(h2, agg3, W3_1, b3_1)
    out = _kb(u3, s3, g3, bt3, W3_2, b3_2, last=True)

    return out.reshape(H)


# speedup vs baseline: 2.4342x; 1.4343x over previous
import functools

import jax
import jax.numpy as jnp
from jax import lax
from jax.experimental import pallas as pl
from jax.experimental.pallas import tpu as pltpu
from jax.experimental.pallas import tpu_sc as plsc

N = 50000
E = 800000
H = 128
NP = 50176
BR = 512
NG = NP // BR

NW = 32
NBUCK = 64
BWN = 800
CAP = 20000
ECH = 4000
K = 64
SUP = 1024
SINK = 800
ZR = 89

_mesh = lambda: plsc.VectorSubcoreMesh(core_axis_name="c", subcore_axis_name="s")
_sc_params = pltpu.CompilerParams(needs_layout_passes=False)


def _wid():
    return lax.axis_index("s") * 2 + lax.axis_index("c")


@functools.partial(
    pl.kernel,
    out_type=(
        jax.ShapeDtypeStruct((NBUCK * CAP,), jnp.int32),
        jax.ShapeDtypeStruct((NBUCK * CAP,), jnp.int32),
        jax.ShapeDtypeStruct((NBUCK * 16,), jnp.int32),
    ),
    mesh=_mesh(),
    compiler_params=_sc_params,
    scratch_types=[
        pltpu.VMEM((ECH,), jnp.int32),
        pltpu.VMEM((ECH,), jnp.int32),
        pltpu.VMEM((CAP,), jnp.int32),
        pltpu.VMEM((CAP,), jnp.int32),
        pltpu.VMEM((CAP,), jnp.int32),
        pltpu.VMEM((CAP,), jnp.int32),
        pltpu.VMEM((16,), jnp.int32),
    ],
)
def _partition(src_hbm, dst_hbm, psrc, pdloc, pcnt,
               sbuf, dbuf, os0, od0, os1, od1, cntbuf):
    w = _wid()
    lo0 = w * BWN
    lo1 = (w + NW) * BWN

    def chunk(c, carry):
        o0, o1 = carry
        pltpu.sync_copy(src_hbm.at[pl.ds(c * ECH, ECH)], sbuf)
        pltpu.sync_copy(dst_hbm.at[pl.ds(c * ECH, ECH)], dbuf)

        def vec(i, carry):
            o0, o1 = carry
            d = dbuf[pl.ds(i * 16, 16)]
            s = sbuf[pl.ds(i * 16, 16)]
            m0 = (d >= lo0) & (d < lo0 + BWN)
            m1 = (d >= lo1) & (d < lo1 + BWN)
            pc0 = plsc.cumsum(m0.astype(jnp.int32))
            pc1 = plsc.cumsum(m1.astype(jnp.int32))
            i0 = o0 + pc0 - 1
            i1 = o1 + pc1 - 1
            plsc.store_scatter(od0, [i0], d - lo0, mask=m0)
            plsc.store_scatter(os0, [i0], s, mask=m0)
            plsc.store_scatter(od1, [i1], d - lo1, mask=m1)
            plsc.store_scatter(os1, [i1], s, mask=m1)
            return o0 + pc0[15], o1 + pc1[15]

        return lax.fori_loop(0, ECH // 16, vec, (o0, o1), unroll=8)

    o0, o1 = lax.fori_loop(0, E // ECH, chunk, (jnp.int32(0), jnp.int32(0)))

    sink_d = jnp.full((16,), SINK, jnp.int32)
    iota16 = lax.iota(jnp.int32, 16)
    for j, (osb, odb, o) in enumerate(((os0, od0, o0), (os1, od1, o1))):
        b = w + NW * j
        for t in range(SUP // 16):
            odb[pl.ds(o + t * 16, 16)] = sink_d
            osb[pl.ds(o + t * 16, 16)] = w * 1536 + t * 16 + iota16
        cnt_pad = ((o + SUP - 1) // SUP) * SUP
        cntbuf[...] = jnp.full((16,), 0, jnp.int32) + cnt_pad
        pltpu.sync_copy(osb, psrc.at[pl.ds(b * CAP, CAP)])
        pltpu.sync_copy(odb, pdloc.at[pl.ds(b * CAP, CAP)])
        pltpu.sync_copy(cntbuf, pcnt.at[pl.ds(b * 16, 16)])


@functools.partial(
    pl.kernel,
    out_type=jax.ShapeDtypeStruct((NP, H), jnp.float32),
    mesh=_mesh(),
    compiler_params=_sc_params,
    scratch_types=[
        pltpu.VMEM((SUP,), jnp.int32),
        pltpu.VMEM((SUP,), jnp.int32),
        pltpu.VMEM((K, H), jnp.float32),
        pltpu.VMEM((K, H), jnp.float32),
        pltpu.VMEM((SINK + 1, H), jnp.float32),
        pltpu.VMEM((16,), jnp.int32),
        pltpu.SemaphoreType.DMA,
        pltpu.SemaphoreType.DMA,
    ],
)
def _agg(h_hbm, psrc, pdloc, pcnt, out_hbm, isrc, idloc, rows0, rows1, acc, cntv, semA, semB):
    w = _wid()
    zero = jnp.zeros((16,), jnp.float32)
    iota = lax.iota(jnp.int32, 16)

    def gather(q16, rbuf, sem):
        return pltpu.async_copy(h_hbm.at[isrc.at[pl.ds(q16, K)]], rbuf, sem)

    def gwait(rbuf, sem):
        pltpu.make_async_copy(h_hbm.at[isrc.at[pl.ds(0, K)]], rbuf, sem).wait()

    def accumulate(q16, rbuf):
        def edge(e, _):
            dsp = plsc.load_gather(idloc, [jnp.full((16,), 0, jnp.int32) + (q16 + e)])
            for f in range(H // 16):
                val = rbuf[e, pl.ds(f * 16, 16)]
                plsc.addupdate_scatter(acc, [dsp, f * 16 + iota], val)
            return 0

        lax.fori_loop(0, K, edge, 0, unroll=4)

    for j in range(2):
        b = w + NW * j

        def zrow(r, _):
            for f in range(H // 16):
                acc[r, pl.ds(f * 16, 16)] = zero
            return 0

        lax.fori_loop(0, SINK + 1, zrow, 0, unroll=4)

        pltpu.sync_copy(pcnt.at[pl.ds(b * 16, 16)], cntv)
        trips = jnp.max(cntv[...]) // SUP

        def sup_body(ts, _):
            pltpu.sync_copy(psrc.at[pl.ds(b * CAP + ts * SUP, SUP)], isrc)
            pltpu.sync_copy(pdloc.at[pl.ds(b * CAP + ts * SUP, SUP)], idloc)
            gather(0, rows0, semA)

            def pair(p, _):
                gather(p * 2 * K + K, rows1, semB)
                gwait(rows0, semA)
                accumulate(p * 2 * K, rows0)

                @pl.when(p < (SUP // (2 * K)) - 1)
                def _():
                    gather(p * 2 * K + 2 * K, rows0, semA)

                gwait(rows1, semB)
                accumulate(p * 2 * K + K, rows1)
                return 0

            lax.fori_loop(0, SUP // (2 * K), pair, 0)
            return 0

        lax.fori_loop(0, trips, sup_body, 0)

        @pl.when(b < NBUCK - 2)
        def _():
            pltpu.sync_copy(acc.at[pl.ds(0, BWN)],
                            out_hbm.at[pl.ds(b * BWN, BWN)])

        @pl.when(b == NBUCK - 2)
        def _():
            pltpu.sync_copy(acc.at[pl.ds(0, N - (NBUCK - 2) * BWN)],
                            out_hbm.at[pl.ds((NBUCK - 2) * BWN,
                                             N - (NBUCK - 2) * BWN)])


def _pre_body(xin, wa, y_ref):
    y_ref[...] = jnp.dot(xin[...], wa[...], preferred_element_type=jnp.float32)


def _pre(xin, wa):
    return pl.pallas_call(
        _pre_body,
        grid=(NG,),
        in_specs=[
            pl.BlockSpec((BR, 16), lambda i: (i, 0)),
            pl.BlockSpec((16, H), lambda i: (0, 0)),
        ],
        out_specs=pl.BlockSpec((BR, H), lambda i: (i, 0)),
        out_shape=jax.ShapeDtypeStruct((NP, H), jnp.float32),
    )(xin, wa)


def _stats_tail(i, u, u_ref, sums_ref):
    rows = i * BR + lax.broadcasted_iota(jnp.int32, (BR, 1), 0)
    u = jnp.where(rows < N, u, 0.0)
    u_ref[...] = u

    @pl.when(i == 0)
    def _():
        sums_ref[...] = jnp.zeros_like(sums_ref)

    sums_ref[0:1, :] += jnp.sum(u, axis=0, keepdims=True)
    sums_ref[1:2, :] += jnp.sum(u * u, axis=0, keepdims=True)


def _ka_body(xin, aggr, wa, ba, u_ref, sums_ref):
    i = pl.program_id(0)
    xa = xin[...] + aggr[...]
    u = jnp.dot(xa, wa[...], preferred_element_type=jnp.float32) + ba[...]
    _stats_tail(i, u, u_ref, sums_ref)


def _ka_add_body(y0, aggr, ba, u_ref, sums_ref):
    i = pl.program_id(0)
    u = y0[...] + aggr[...] + ba[...]
    _stats_tail(i, u, u_ref, sums_ref)


_KA_OUT = [
    jax.ShapeDtypeStruct((NP, H), jnp.float32),
    jax.ShapeDtypeStruct((2, H), jnp.float32),
]
_KA_OUT_SPECS = [
    pl.BlockSpec((BR, H), lambda i: (i, 0)),
    pl.BlockSpec((2, H), lambda i: (0, 0)),
]


def _ka(xin, aggr, wa, ba):
    return pl.pallas_call(
        _ka_body,
        grid=(NG,),
        in_specs=[
            pl.BlockSpec((BR, H), lambda i: (i, 0)),
            pl.BlockSpec((BR, H), lambda i: (i, 0)),
            pl.BlockSpec((H, H), lambda i: (0, 0)),
            pl.BlockSpec((1, H), lambda i: (0, 0)),
        ],
        out_specs=_KA_OUT_SPECS,
        out_shape=_KA_OUT,
    )(xin, aggr, wa, ba.reshape(1, H))


def _ka_add(y0, aggr, ba):
    return pl.pallas_call(
        _ka_add_body,
        grid=(NG,),
        in_specs=[
            pl.BlockSpec((BR, H), lambda i: (i, 0)),
            pl.BlockSpec((BR, H), lambda i: (i, 0)),
            pl.BlockSpec((1, H), lambda i: (0, 0)),
        ],
        out_specs=_KA_OUT_SPECS,
        out_shape=_KA_OUT,
    )(y0, aggr, ba.reshape(1, H))


def _bn_relu(u, sums, g, bt):
    m = sums[0:1, :] * (1.0 / N)
    var = sums[1:2, :] * (1.0 / N) - m * m
    inv = g[...] * lax.rsqrt(var + 1e-5)
    return jnp.maximum(u[...] * inv + (bt[...] - m * inv), 0.0)


def _kb_body(u, sums, g, bt, wb, bb, h_ref):
    t = _bn_relu(u, sums, g, bt)
    h_ref[...] = jnp.maximum(
        jnp.dot(t, wb[...], preferred_element_type=jnp.float32) + bb[...], 0.0)


def _kb3_body(u, sums, g, bt, wb, bb, out_ref):
    i = pl.program_id(0)
    t = _bn_relu(u, sums, g, bt)
    h = jnp.maximum(
        jnp.dot(t, wb[...], preferred_element_type=jnp.float32) + bb[...], 0.0)
    rows = i * BR + lax.broadcasted_iota(jnp.int32, (BR, 1), 0)
    h = jnp.where(rows < N, h, 0.0)

    @pl.when(i == 0)
    def _():
        out_ref[...] = jnp.zeros_like(out_ref)

    out_ref[...] += jnp.sum(h, axis=0, keepdims=True)


def _kb(u, sums, g, bt, wb, bb, last):
    in_specs = [
        pl.BlockSpec((BR, H), lambda i: (i, 0)),
        pl.BlockSpec((2, H), lambda i: (0, 0)),
        pl.BlockSpec((1, H), lambda i: (0, 0)),
        pl.BlockSpec((1, H), lambda i: (0, 0)),
        pl.BlockSpec((H, H), lambda i: (0, 0)),
        pl.BlockSpec((1, H), lambda i: (0, 0)),
    ]
    if last:
        out_spec = pl.BlockSpec((1, H), lambda i: (0, 0))
        out_shape = jax.ShapeDtypeStruct((1, H), jnp.float32)
        body = _kb3_body
    else:
        out_spec = pl.BlockSpec((BR, H), lambda i: (i, 0))
        out_shape = jax.ShapeDtypeStruct((NP, H), jnp.float32)
        body = _kb_body
    return pl.pallas_call(
        body, grid=(NG,), in_specs=in_specs, out_specs=out_spec,
        out_shape=out_shape,
    )(u, sums, g.reshape(1, H), bt.reshape(1, H), wb, bb.reshape(1, H))


def kernel(x, edge_index,
           W1_1, b1_1, g1, bt1, W1_2, b1_2,
           W2_1, b2_1, g2, bt2, W2_2, b2_2,
           W3_1, b3_1, g3, bt3, W3_2, b3_2):
    src = edge_index[0]
    dst = edge_index[1]

    x_p = jnp.zeros((NP, 16), jnp.float32).at[:N, :9].set(x)
    w1_p = jnp.zeros((16, H), jnp.float32).at[:9, :].set(W1_1)

    psrc, pdloc, pcnt = _partition(src, dst)

    y0 = _pre(x_p, w1_p)
    agg1 = _agg(y0, psrc, pdloc, pcnt)
    u1, s1 = _ka_add(y0, agg1, b1_1)
    h1 = _kb(u1, s1, g1, bt1, W1_2, b1_2, last=False)

    agg2 = _agg(h1, psrc, pdloc, pcnt)
    u2, s2 = _ka(h1, agg2, W2_1, b2_1)
    h2 = _kb(u2, s2, g2, bt2, W2_2, b2_2, last=False)

    agg3 = _agg(h2, psrc, pdloc, pcnt)
    u3, s3 = _ka(h2, agg3, W3_1, b3_1)
    out = _kb(u3, s3, g3, bt3, W3_2, b3_2, last=True)

    return out.reshape(H)


# speedup vs baseline: 2.4720x; 1.0156x over previous
import functools

import jax
import jax.numpy as jnp
from jax import lax
from jax.experimental import pallas as pl
from jax.experimental.pallas import tpu as pltpu
from jax.experimental.pallas import tpu_sc as plsc

N = 50000
E = 800000
H = 128
NP = 50176
BR = 512
NG = NP // BR

NW = 32
NBUCK = 64
BWN = 800
CAP = 20000
ECH = 4000
K = 64
SUP = 1024
SINK = 800
ZR = 89

_mesh = lambda: plsc.VectorSubcoreMesh(core_axis_name="c", subcore_axis_name="s")
_sc_params = pltpu.CompilerParams(needs_layout_passes=False)


def _wid():
    return lax.axis_index("s") * 2 + lax.axis_index("c")


@functools.partial(
    pl.kernel,
    out_type=(
        jax.ShapeDtypeStruct((NBUCK * CAP,), jnp.int32),
        jax.ShapeDtypeStruct((NBUCK * CAP,), jnp.int32),
        jax.ShapeDtypeStruct((NBUCK * 16,), jnp.int32),
    ),
    mesh=_mesh(),
    compiler_params=_sc_params,
    scratch_types=[
        pltpu.VMEM((ECH,), jnp.int32),
        pltpu.VMEM((ECH,), jnp.int32),
        pltpu.VMEM((CAP,), jnp.int32),
        pltpu.VMEM((CAP,), jnp.int32),
        pltpu.VMEM((CAP,), jnp.int32),
        pltpu.VMEM((CAP,), jnp.int32),
        pltpu.VMEM((16,), jnp.int32),
    ],
)
def _partition(src_hbm, dst_hbm, psrc, pdloc, pcnt,
               sbuf, dbuf, os0, od0, os1, od1, cntbuf):
    w = _wid()
    lo0 = w * BWN
    lo1 = (w + NW) * BWN

    def chunk(c, carry):
        o0, o1 = carry
        pltpu.sync_copy(src_hbm.at[pl.ds(c * ECH, ECH)], sbuf)
        pltpu.sync_copy(dst_hbm.at[pl.ds(c * ECH, ECH)], dbuf)

        def vec(i, carry):
            o0, o1 = carry
            d = dbuf[pl.ds(i * 16, 16)]
            s = sbuf[pl.ds(i * 16, 16)]
            m0 = (d >= lo0) & (d < lo0 + BWN)
            m1 = (d >= lo1) & (d < lo1 + BWN)
            n0 = plsc.all_reduce_population_count(m0)
            n1 = plsc.all_reduce_population_count(m1)
            pc0 = plsc.cumsum(m0.astype(jnp.int32))
            pc1 = plsc.cumsum(m1.astype(jnp.int32))
            i0 = o0 + pc0 - 1
            i1 = o1 + pc1 - 1
            plsc.store_scatter(od0, [i0], d - lo0, mask=m0)
            plsc.store_scatter(os0, [i0], s, mask=m0)
            plsc.store_scatter(od1, [i1], d - lo1, mask=m1)
            plsc.store_scatter(os1, [i1], s, mask=m1)
            return o0 + n0, o1 + n1

        return lax.fori_loop(0, ECH // 16, vec, (o0, o1), unroll=8)

    zv = jnp.zeros((16,), jnp.int32)
    o0v, o1v = lax.fori_loop(0, E // ECH, chunk, (zv, zv))
    o0 = o0v[0]
    o1 = o1v[0]

    sink_d = jnp.full((16,), SINK, jnp.int32)
    iota16 = lax.iota(jnp.int32, 16)
    for j, (osb, odb, o) in enumerate(((os0, od0, o0), (os1, od1, o1))):
        b = w + NW * j
        for t in range(SUP // 16):
            odb[pl.ds(o + t * 16, 16)] = sink_d
            osb[pl.ds(o + t * 16, 16)] = w * 1536 + t * 16 + iota16
        cnt_pad = ((o + SUP - 1) // SUP) * SUP
        cntbuf[...] = jnp.full((16,), 0, jnp.int32) + cnt_pad
        pltpu.sync_copy(osb, psrc.at[pl.ds(b * CAP, CAP)])
        pltpu.sync_copy(odb, pdloc.at[pl.ds(b * CAP, CAP)])
        pltpu.sync_copy(cntbuf, pcnt.at[pl.ds(b * 16, 16)])


@functools.partial(
    pl.kernel,
    out_type=jax.ShapeDtypeStruct((NP, H), jnp.float32),
    mesh=_mesh(),
    compiler_params=_sc_params,
    scratch_types=[
        pltpu.VMEM((SUP,), jnp.int32),
        pltpu.VMEM((SUP,), jnp.int32),
        pltpu.VMEM((K, H), jnp.float32),
        pltpu.VMEM((K, H), jnp.float32),
        pltpu.VMEM((SINK + 1, H), jnp.float32),
        pltpu.VMEM((16,), jnp.int32),
        pltpu.SemaphoreType.DMA,
        pltpu.SemaphoreType.DMA,
    ],
)
def _agg(h_hbm, psrc, pdloc, pcnt, out_hbm, isrc, idloc, rows0, rows1, acc, cntv, semA, semB):
    w = _wid()
    zero = jnp.zeros((16,), jnp.float32)
    iota = lax.iota(jnp.int32, 16)

    def gather(q16, rbuf, sem):
        return pltpu.async_copy(h_hbm.at[isrc.at[pl.ds(q16, K)]], rbuf, sem)

    def gwait(rbuf, sem):
        pltpu.make_async_copy(h_hbm.at[isrc.at[pl.ds(0, K)]], rbuf, sem).wait()

    def accumulate(q16, rbuf):
        def edge(e, _):
            dsp = plsc.load_gather(idloc, [jnp.full((16,), 0, jnp.int32) + (q16 + e)])
            for f in range(H // 16):
                val = rbuf[e, pl.ds(f * 16, 16)]
                plsc.addupdate_scatter(acc, [dsp, f * 16 + iota], val)
            return 0

        lax.fori_loop(0, K, edge, 0, unroll=8)

    for j in range(2):
        b = w + NW * j

        def zrow(r, _):
            for f in range(H // 16):
                acc[r, pl.ds(f * 16, 16)] = zero
            return 0

        lax.fori_loop(0, SINK + 1, zrow, 0, unroll=4)

        pltpu.sync_copy(pcnt.at[pl.ds(b * 16, 16)], cntv)
        trips = jnp.max(cntv[...]) // SUP

        def sup_body(ts, _):
            pltpu.sync_copy(psrc.at[pl.ds(b * CAP + ts * SUP, SUP)], isrc)
            pltpu.sync_copy(pdloc.at[pl.ds(b * CAP + ts * SUP, SUP)], idloc)
            gather(0, rows0, semA)

            def pair(p, _):
                gather(p * 2 * K + K, rows1, semB)
                gwait(rows0, semA)
                accumulate(p * 2 * K, rows0)

                @pl.when(p < (SUP // (2 * K)) - 1)
                def _():
                    gather(p * 2 * K + 2 * K, rows0, semA)

                gwait(rows1, semB)
                accumulate(p * 2 * K + K, rows1)
                return 0

            lax.fori_loop(0, SUP // (2 * K), pair, 0)
            return 0

        lax.fori_loop(0, trips, sup_body, 0)

        @pl.when(b < NBUCK - 2)
        def _():
            pltpu.sync_copy(acc.at[pl.ds(0, BWN)],
                            out_hbm.at[pl.ds(b * BWN, BWN)])

        @pl.when(b == NBUCK - 2)
        def _():
            pltpu.sync_copy(acc.at[pl.ds(0, N - (NBUCK - 2) * BWN)],
                            out_hbm.at[pl.ds((NBUCK - 2) * BWN,
                                             N - (NBUCK - 2) * BWN)])


def _pre_body(xin, wa, y_ref):
    y_ref[...] = jnp.dot(xin[...], wa[...], preferred_element_type=jnp.float32)


def _pre(xin, wa):
    return pl.pallas_call(
        _pre_body,
        grid=(NG,),
        in_specs=[
            pl.BlockSpec((BR, 16), lambda i: (i, 0)),
            pl.BlockSpec((16, H), lambda i: (0, 0)),
        ],
        out_specs=pl.BlockSpec((BR, H), lambda i: (i, 0)),
        out_shape=jax.ShapeDtypeStruct((NP, H), jnp.float32),
    )(xin, wa)


def _stats_tail(i, u, u_ref, sums_ref):
    rows = i * BR + lax.broadcasted_iota(jnp.int32, (BR, 1), 0)
    u = jnp.where(rows < N, u, 0.0)
    u_ref[...] = u

    @pl.when(i == 0)
    def _():
        sums_ref[...] = jnp.zeros_like(sums_ref)

    sums_ref[0:1, :] += jnp.sum(u, axis=0, keepdims=True)
    sums_ref[1:2, :] += jnp.sum(u * u, axis=0, keepdims=True)


def _ka_body(xin, aggr, wa, ba, u_ref, sums_ref):
    i = pl.program_id(0)
    xa = xin[...] + aggr[...]
    u = jnp.dot(xa, wa[...], preferred_element_type=jnp.float32) + ba[...]
    _stats_tail(i, u, u_ref, sums_ref)


def _ka_add_body(y0, aggr, ba, u_ref, sums_ref):
    i = pl.program_id(0)
    u = y0[...] + aggr[...] + ba[...]
    _stats_tail(i, u, u_ref, sums_ref)


_KA_OUT = [
    jax.ShapeDtypeStruct((NP, H), jnp.float32),
    jax.ShapeDtypeStruct((2, H), jnp.float32),
]
_KA_OUT_SPECS = [
    pl.BlockSpec((BR, H), lambda i: (i, 0)),
    pl.BlockSpec((2, H), lambda i: (0, 0)),
]


def _ka(xin, aggr, wa, ba):
    return pl.pallas_call(
        _ka_body,
        grid=(NG,),
        in_specs=[
            pl.BlockSpec((BR, H), lambda i: (i, 0)),
            pl.BlockSpec((BR, H), lambda i: (i, 0)),
            pl.BlockSpec((H, H), lambda i: (0, 0)),
            pl.BlockSpec((1, H), lambda i: (0, 0)),
        ],
        out_specs=_KA_OUT_SPECS,
        out_shape=_KA_OUT,
    )(xin, aggr, wa, ba.reshape(1, H))


def _ka_add(y0, aggr, ba):
    return pl.pallas_call(
        _ka_add_body,
        grid=(NG,),
        in_specs=[
            pl.BlockSpec((BR, H), lambda i: (i, 0)),
            pl.BlockSpec((BR, H), lambda i: (i, 0)),
            pl.BlockSpec((1, H), lambda i: (0, 0)),
        ],
        out_specs=_KA_OUT_SPECS,
        out_shape=_KA_OUT,
    )(y0, aggr, ba.reshape(1, H))


def _bn_relu(u, sums, g, bt):
    m = sums[0:1, :] * (1.0 / N)
    var = sums[1:2, :] * (1.0 / N) - m * m
    inv = g[...] * lax.rsqrt(var + 1e-5)
    return jnp.maximum(u[...] * inv + (bt[...] - m * inv), 0.0)


def _kb_body(u, sums, g, bt, wb, bb, h_ref):
    t = _bn_relu(u, sums, g, bt)
    h_ref[...] = jnp.maximum(
        jnp.dot(t, wb[...], preferred_element_type=jnp.float32) + bb[...], 0.0)


def _kb3_body(u, sums, g, bt, wb, bb, out_ref):
    i = pl.program_id(0)
    t = _bn_relu(u, sums, g, bt)
    h = jnp.maximum(
        jnp.dot(t, wb[...], preferred_element_type=jnp.float32) + bb[...], 0.0)
    rows = i * BR + lax.broadcasted_iota(jnp.int32, (BR, 1), 0)
    h = jnp.where(rows < N, h, 0.0)

    @pl.when(i == 0)
    def _():
        out_ref[...] = jnp.zeros_like(out_ref)

    out_ref[...] += jnp.sum(h, axis=0, keepdims=True)


def _kb(u, sums, g, bt, wb, bb, last):
    in_specs = [
        pl.BlockSpec((BR, H), lambda i: (i, 0)),
        pl.BlockSpec((2, H), lambda i: (0, 0)),
        pl.BlockSpec((1, H), lambda i: (0, 0)),
        pl.BlockSpec((1, H), lambda i: (0, 0)),
        pl.BlockSpec((H, H), lambda i: (0, 0)),
        pl.BlockSpec((1, H), lambda i: (0, 0)),
    ]
    if last:
        out_spec = pl.BlockSpec((1, H), lambda i: (0, 0))
        out_shape = jax.ShapeDtypeStruct((1, H), jnp.float32)
        body = _kb3_body
    else:
        out_spec = pl.BlockSpec((BR, H), lambda i: (i, 0))
        out_shape = jax.ShapeDtypeStruct((NP, H), jnp.float32)
        body = _kb_body
    return pl.pallas_call(
        body, grid=(NG,), in_specs=in_specs, out_specs=out_spec,
        out_shape=out_shape,
    )(u, sums, g.reshape(1, H), bt.reshape(1, H), wb, bb.reshape(1, H))


def kernel(x, edge_index,
           W1_1, b1_1, g1, bt1, W1_2, b1_2,
           W2_1, b2_1, g2, bt2, W2_2, b2_2,
           W3_1, b3_1, g3, bt3, W3_2, b3_2):
    src = edge_index[0]
    dst = edge_index[1]

    x_p = jnp.zeros((NP, 16), jnp.float32).at[:N, :9].set(x)
    w1_p = jnp.zeros((16, H), jnp.float32).at[:9, :].set(W1_1)

    psrc, pdloc, pcnt = _partition(src, dst)

    y0 = _pre(x_p, w1_p)
    agg1 = _agg(y0, psrc, pdloc, pcnt)
    u1, s1 = _ka_add(y0, agg1, b1_1)
    h1 = _kb(u1, s1, g1, bt1, W1_2, b1_2, last=False)

    agg2 = _agg(h1, psrc, pdloc, pcnt)
    u2, s2 = _ka(h1, agg2, W2_1, b2_1)
    h2 = _kb(u2, s2, g2, bt2, W2_2, b2_2, last=False)

    agg3 = _agg(h2, psrc, pdloc, pcnt)
    u3, s3 = _ka(h2, agg3, W3_1, b3_1)
    out = _kb(u3, s3, g3, bt3, W3_2, b3_2, last=True)

    return out.reshape(H)


# speedup vs baseline: 2.5990x; 1.0514x over previous
import functools

import jax
import jax.numpy as jnp
from jax import lax
from jax.experimental import pallas as pl
from jax.experimental.pallas import tpu as pltpu
from jax.experimental.pallas import tpu_sc as plsc

N = 50000
E = 800000
H = 128
NP = 50176
BR = 512
NG = NP // BR

NW = 32
NBUCK = 64
BWN = 800
CAP = 20000
ECH = 4000
K = 64
SUP = 1024
SINK = 800
ZR = 89

_mesh = lambda: plsc.VectorSubcoreMesh(core_axis_name="c", subcore_axis_name="s")
_sc_params = pltpu.CompilerParams(needs_layout_passes=False)


def _wid():
    return lax.axis_index("s") * 2 + lax.axis_index("c")


@functools.partial(
    pl.kernel,
    out_type=(
        jax.ShapeDtypeStruct((NBUCK * CAP,), jnp.int32),
        jax.ShapeDtypeStruct((NBUCK * CAP,), jnp.int32),
        jax.ShapeDtypeStruct((NBUCK * 16,), jnp.int32),
    ),
    mesh=_mesh(),
    compiler_params=_sc_params,
    scratch_types=[
        pltpu.VMEM((ECH,), jnp.int32),
        pltpu.VMEM((ECH,), jnp.int32),
        pltpu.VMEM((CAP,), jnp.int32),
        pltpu.VMEM((CAP,), jnp.int32),
        pltpu.VMEM((CAP,), jnp.int32),
        pltpu.VMEM((CAP,), jnp.int32),
        pltpu.VMEM((16,), jnp.int32),
    ],
)
def _partition(src_hbm, dst_hbm, psrc, pdloc, pcnt,
               sbuf, dbuf, os0, od0, os1, od1, cntbuf):
    w = _wid()
    lo0 = w * BWN
    lo1 = (w + NW) * BWN

    def chunk(c, carry):
        o0, o1 = carry
        pltpu.sync_copy(src_hbm.at[pl.ds(c * ECH, ECH)], sbuf)
        pltpu.sync_copy(dst_hbm.at[pl.ds(c * ECH, ECH)], dbuf)

        def vec(i, carry):
            o0, o1 = carry
            d = dbuf[pl.ds(i * 16, 16)]
            s = sbuf[pl.ds(i * 16, 16)]
            m0 = (d >= lo0) & (d < lo0 + BWN)
            m1 = (d >= lo1) & (d < lo1 + BWN)
            n0 = plsc.all_reduce_population_count(m0)
            n1 = plsc.all_reduce_population_count(m1)
            pc0 = plsc.cumsum(m0.astype(jnp.int32))
            pc1 = plsc.cumsum(m1.astype(jnp.int32))
            i0 = o0 + pc0 - 1
            i1 = o1 + pc1 - 1
            plsc.store_scatter(od0, [i0], d - lo0, mask=m0)
            plsc.store_scatter(os0, [i0], s, mask=m0)
            plsc.store_scatter(od1, [i1], d - lo1, mask=m1)
            plsc.store_scatter(os1, [i1], s, mask=m1)
            return o0 + n0, o1 + n1

        return lax.fori_loop(0, ECH // 16, vec, (o0, o1), unroll=8)

    zv = jnp.zeros((16,), jnp.int32)
    o0v, o1v = lax.fori_loop(0, E // ECH, chunk, (zv, zv))
    o0 = o0v[0]
    o1 = o1v[0]

    sink_d = jnp.full((16,), SINK, jnp.int32)
    iota16 = lax.iota(jnp.int32, 16)
    for j, (osb, odb, o) in enumerate(((os0, od0, o0), (os1, od1, o1))):
        b = w + NW * j
        for t in range(SUP // 16):
            odb[pl.ds(o + t * 16, 16)] = sink_d
            osb[pl.ds(o + t * 16, 16)] = w * 1536 + t * 16 + iota16
        cnt_pad = ((o + SUP - 1) // SUP) * SUP
        cntbuf[...] = jnp.full((16,), 0, jnp.int32) + cnt_pad
        pltpu.sync_copy(osb, psrc.at[pl.ds(b * CAP, CAP)])
        pltpu.sync_copy(odb, pdloc.at[pl.ds(b * CAP, CAP)])
        pltpu.sync_copy(cntbuf, pcnt.at[pl.ds(b * 16, 16)])


@functools.partial(
    pl.kernel,
    out_type=jax.ShapeDtypeStruct((NP, H), jnp.float32),
    mesh=_mesh(),
    compiler_params=_sc_params,
    scratch_types=[
        pltpu.VMEM((SUP,), jnp.int32),
        pltpu.VMEM((SUP,), jnp.int32),
        pltpu.SMEM((K,), jnp.int32),
        pltpu.VMEM_SHARED((16, SUP), jnp.int32),
        pltpu.VMEM((K, H), jnp.float32),
        pltpu.VMEM((K, H), jnp.float32),
        pltpu.VMEM((SINK + 1, H), jnp.float32),
        pltpu.VMEM((16,), jnp.int32),
        pltpu.SemaphoreType.DMA,
        pltpu.SemaphoreType.DMA,
    ],
)
def _agg(h_hbm, psrc, pdloc, pcnt, out_hbm, isrc, idloc, sdloc, bdloc, rows0, rows1, acc, cntv, semA, semB):
    w = _wid()
    zero = jnp.zeros((16,), jnp.float32)
    iota = lax.iota(jnp.int32, 16)

    def gather(q16, rbuf, sem):
        return pltpu.async_copy(h_hbm.at[isrc.at[pl.ds(q16, K)]], rbuf, sem)

    def gwait(rbuf, sem):
        pltpu.make_async_copy(h_hbm.at[isrc.at[pl.ds(0, K)]], rbuf, sem).wait()

    def accumulate(q16, rbuf):
        sid = lax.axis_index("s")
        pltpu.sync_copy(bdloc.at[sid, pl.ds(q16, K)], sdloc)

        def edge(e, _):
            ds_row = sdloc[e]
            for f in range(H // 16):
                val = rbuf[e, pl.ds(f * 16, 16)]
                plsc.addupdate(acc.at[ds_row, pl.ds(f * 16, 16)], val)
            return 0

        lax.fori_loop(0, K, edge, 0, unroll=8)

    for j in range(2):
        b = w + NW * j

        def zrow(r, _):
            for f in range(H // 16):
                acc[r, pl.ds(f * 16, 16)] = zero
            return 0

        lax.fori_loop(0, SINK + 1, zrow, 0, unroll=4)

        pltpu.sync_copy(pcnt.at[pl.ds(b * 16, 16)], cntv)
        trips = jnp.max(cntv[...]) // SUP

        def sup_body(ts, _):
            pltpu.sync_copy(psrc.at[pl.ds(b * CAP + ts * SUP, SUP)], isrc)
            pltpu.sync_copy(pdloc.at[pl.ds(b * CAP + ts * SUP, SUP)], idloc)
            pltpu.sync_copy(idloc, bdloc.at[lax.axis_index("s")])
            gather(0, rows0, semA)

            def pair(p, _):
                gather(p * 2 * K + K, rows1, semB)
                gwait(rows0, semA)
                accumulate(p * 2 * K, rows0)

                @pl.when(p < (SUP // (2 * K)) - 1)
                def _():
                    gather(p * 2 * K + 2 * K, rows0, semA)

                gwait(rows1, semB)
                accumulate(p * 2 * K + K, rows1)
                return 0

            lax.fori_loop(0, SUP // (2 * K), pair, 0)
            return 0

        lax.fori_loop(0, trips, sup_body, 0)

        @pl.when(b < NBUCK - 2)
        def _():
            pltpu.sync_copy(acc.at[pl.ds(0, BWN)],
                            out_hbm.at[pl.ds(b * BWN, BWN)])

        @pl.when(b == NBUCK - 2)
        def _():
            pltpu.sync_copy(acc.at[pl.ds(0, N - (NBUCK - 2) * BWN)],
                            out_hbm.at[pl.ds((NBUCK - 2) * BWN,
                                             N - (NBUCK - 2) * BWN)])


def _pre_body(xin, wa, y_ref):
    y_ref[...] = jnp.dot(xin[...], wa[...], preferred_element_type=jnp.float32)


def _pre(xin, wa):
    return pl.pallas_call(
        _pre_body,
        grid=(NG,),
        in_specs=[
            pl.BlockSpec((BR, 16), lambda i: (i, 0)),
            pl.BlockSpec((16, H), lambda i: (0, 0)),
        ],
        out_specs=pl.BlockSpec((BR, H), lambda i: (i, 0)),
        out_shape=jax.ShapeDtypeStruct((NP, H), jnp.float32),
    )(xin, wa)


def _stats_tail(i, u, u_ref, sums_ref):
    rows = i * BR + lax.broadcasted_iota(jnp.int32, (BR, 1), 0)
    u = jnp.where(rows < N, u, 0.0)
    u_ref[...] = u

    @pl.when(i == 0)
    def _():
        sums_ref[...] = jnp.zeros_like(sums_ref)

    sums_ref[0:1, :] += jnp.sum(u, axis=0, keepdims=True)
    sums_ref[1:2, :] += jnp.sum(u * u, axis=0, keepdims=True)


def _ka_body(xin, aggr, wa, ba, u_ref, sums_ref):
    i = pl.program_id(0)
    xa = xin[...] + aggr[...]
    u = jnp.dot(xa, wa[...], preferred_element_type=jnp.float32) + ba[...]
    _stats_tail(i, u, u_ref, sums_ref)


def _ka_add_body(y0, aggr, ba, u_ref, sums_ref):
    i = pl.program_id(0)
    u = y0[...] + aggr[...] + ba[...]
    _stats_tail(i, u, u_ref, sums_ref)


_KA_OUT = [
    jax.ShapeDtypeStruct((NP, H), jnp.float32),
    jax.ShapeDtypeStruct((2, H), jnp.float32),
]
_KA_OUT_SPECS = [
    pl.BlockSpec((BR, H), lambda i: (i, 0)),
    pl.BlockSpec((2, H), lambda i: (0, 0)),
]


def _ka(xin, aggr, wa, ba):
    return pl.pallas_call(
        _ka_body,
        grid=(NG,),
        in_specs=[
            pl.BlockSpec((BR, H), lambda i: (i, 0)),
            pl.BlockSpec((BR, H), lambda i: (i, 0)),
            pl.BlockSpec((H, H), lambda i: (0, 0)),
            pl.BlockSpec((1, H), lambda i: (0, 0)),
        ],
        out_specs=_KA_OUT_SPECS,
        out_shape=_KA_OUT,
    )(xin, aggr, wa, ba.reshape(1, H))


def _ka_add(y0, aggr, ba):
    return pl.pallas_call(
        _ka_add_body,
        grid=(NG,),
        in_specs=[
            pl.BlockSpec((BR, H), lambda i: (i, 0)),
            pl.BlockSpec((BR, H), lambda i: (i, 0)),
            pl.BlockSpec((1, H), lambda i: (0, 0)),
        ],
        out_specs=_KA_OUT_SPECS,
        out_shape=_KA_OUT,
    )(y0, aggr, ba.reshape(1, H))


def _bn_relu(u, sums, g, bt):
    m = sums[0:1, :] * (1.0 / N)
    var = sums[1:2, :] * (1.0 / N) - m * m
    inv = g[...] * lax.rsqrt(var + 1e-5)
    return jnp.maximum(u[...] * inv + (bt[...] - m * inv), 0.0)


def _kb_body(u, sums, g, bt, wb, bb, h_ref):
    t = _bn_relu(u, sums, g, bt)
    h_ref[...] = jnp.maximum(
        jnp.dot(t, wb[...], preferred_element_type=jnp.float32) + bb[...], 0.0)


def _kb3_body(u, sums, g, bt, wb, bb, out_ref):
    i = pl.program_id(0)
    t = _bn_relu(u, sums, g, bt)
    h = jnp.maximum(
        jnp.dot(t, wb[...], preferred_element_type=jnp.float32) + bb[...], 0.0)
    rows = i * BR + lax.broadcasted_iota(jnp.int32, (BR, 1), 0)
    h = jnp.where(rows < N, h, 0.0)

    @pl.when(i == 0)
    def _():
        out_ref[...] = jnp.zeros_like(out_ref)

    out_ref[...] += jnp.sum(h, axis=0, keepdims=True)


def _kb(u, sums, g, bt, wb, bb, last):
    in_specs = [
        pl.BlockSpec((BR, H), lambda i: (i, 0)),
        pl.BlockSpec((2, H), lambda i: (0, 0)),
        pl.BlockSpec((1, H), lambda i: (0, 0)),
        pl.BlockSpec((1, H), lambda i: (0, 0)),
        pl.BlockSpec((H, H), lambda i: (0, 0)),
        pl.BlockSpec((1, H), lambda i: (0, 0)),
    ]
    if last:
        out_spec = pl.BlockSpec((1, H), lambda i: (0, 0))
        out_shape = jax.ShapeDtypeStruct((1, H), jnp.float32)
        body = _kb3_body
    else:
        out_spec = pl.BlockSpec((BR, H), lambda i: (i, 0))
        out_shape = jax.ShapeDtypeStruct((NP, H), jnp.float32)
        body = _kb_body
    return pl.pallas_call(
        body, grid=(NG,), in_specs=in_specs, out_specs=out_spec,
        out_shape=out_shape,
    )(u, sums, g.reshape(1, H), bt.reshape(1, H), wb, bb.reshape(1, H))


def kernel(x, edge_index,
           W1_1, b1_1, g1, bt1, W1_2, b1_2,
           W2_1, b2_1, g2, bt2, W2_2, b2_2,
           W3_1, b3_1, g3, bt3, W3_2, b3_2):
    src = edge_index[0]
    dst = edge_index[1]

    x_p = jnp.zeros((NP, 16), jnp.float32).at[:N, :9].set(x)
    w1_p = jnp.zeros((16, H), jnp.float32).at[:9, :].set(W1_1)

    psrc, pdloc, pcnt = _partition(src, dst)

    y0 = _pre(x_p, w1_p)
    agg1 = _agg(y0, psrc, pdloc, pcnt)
    u1, s1 = _ka_add(y0, agg1, b1_1)
    h1 = _kb(u1, s1, g1, bt1, W1_2, b1_2, last=False)

    agg2 = _agg(h1, psrc, pdloc, pcnt)
    u2, s2 = _ka(h1, agg2, W2_1, b2_1)
    h2 = _kb(u2, s2, g2, bt2, W2_2, b2_2, last=False)

    agg3 = _agg(h2, psrc, pdloc, pcnt)
    u3, s3 = _ka(h2, agg3, W3_1, b3_1)
    out = _kb(u3, s3, g3, bt3, W3_2, b3_2, last=True)

    return out.reshape(H)


# speedup vs baseline: 2.7185x; 1.0460x over previous
import functools

import jax
import jax.numpy as jnp
from jax import lax
from jax.experimental import pallas as pl
from jax.experimental.pallas import tpu as pltpu
from jax.experimental.pallas import tpu_sc as plsc

N = 50000
E = 800000
H = 128
NP = 50176
BR = 512
NG = NP // BR

NW = 32
NBUCK = 64
BWN = 800
CAP = 20000
ECH = 16000
K = 64
SUP = 1024
SINK = 800
ZR = 89

_mesh = lambda: plsc.VectorSubcoreMesh(core_axis_name="c", subcore_axis_name="s")
_sc_params = pltpu.CompilerParams(needs_layout_passes=False)


def _wid():
    return lax.axis_index("s") * 2 + lax.axis_index("c")


@functools.partial(
    pl.kernel,
    out_type=(
        jax.ShapeDtypeStruct((NBUCK * CAP,), jnp.int32),
        jax.ShapeDtypeStruct((NBUCK * CAP,), jnp.int32),
        jax.ShapeDtypeStruct((NBUCK * 16,), jnp.int32),
    ),
    mesh=_mesh(),
    compiler_params=_sc_params,
    scratch_types=[
        pltpu.VMEM((ECH,), jnp.int32),
        pltpu.VMEM((ECH,), jnp.int32),
        pltpu.VMEM((CAP,), jnp.int32),
        pltpu.VMEM((CAP,), jnp.int32),
        pltpu.VMEM((CAP,), jnp.int32),
        pltpu.VMEM((CAP,), jnp.int32),
        pltpu.VMEM((16,), jnp.int32),
    ],
)
def _partition(src_hbm, dst_hbm, psrc, pdloc, pcnt,
               sbuf, dbuf, os0, od0, os1, od1, cntbuf):
    w = _wid()
    lo0 = w * BWN
    lo1 = (w + NW) * BWN

    def chunk(c, carry):
        o0, o1 = carry
        pltpu.sync_copy(src_hbm.at[pl.ds(c * ECH, ECH)], sbuf)
        pltpu.sync_copy(dst_hbm.at[pl.ds(c * ECH, ECH)], dbuf)

        def vec(i, carry):
            o0, o1 = carry
            d = dbuf[pl.ds(i * 16, 16)]
            s = sbuf[pl.ds(i * 16, 16)]
            m0 = (d >= lo0) & (d < lo0 + BWN)
            m1 = (d >= lo1) & (d < lo1 + BWN)
            n0 = plsc.all_reduce_population_count(m0)
            n1 = plsc.all_reduce_population_count(m1)
            pc0 = plsc.cumsum(m0.astype(jnp.int32))
            pc1 = plsc.cumsum(m1.astype(jnp.int32))
            i0 = o0 + pc0 - 1
            i1 = o1 + pc1 - 1
            plsc.store_scatter(od0, [i0], d - lo0, mask=m0)
            plsc.store_scatter(os0, [i0], s, mask=m0)
            plsc.store_scatter(od1, [i1], d - lo1, mask=m1)
            plsc.store_scatter(os1, [i1], s, mask=m1)
            return o0 + n0, o1 + n1

        return lax.fori_loop(0, ECH // 16, vec, (o0, o1), unroll=8)

    zv = jnp.zeros((16,), jnp.int32)
    o0v, o1v = lax.fori_loop(0, E // ECH, chunk, (zv, zv))
    o0 = o0v[0]
    o1 = o1v[0]

    sink_d = jnp.full((16,), SINK, jnp.int32)
    iota16 = lax.iota(jnp.int32, 16)
    for j, (osb, odb, o) in enumerate(((os0, od0, o0), (os1, od1, o1))):
        b = w + NW * j
        for t in range(SUP // 16):
            odb[pl.ds(o + t * 16, 16)] = sink_d
            osb[pl.ds(o + t * 16, 16)] = w * 1536 + t * 16 + iota16
        cnt_pad = ((o + SUP - 1) // SUP) * SUP
        cntbuf[...] = jnp.full((16,), 0, jnp.int32) + cnt_pad
        pltpu.sync_copy(osb, psrc.at[pl.ds(b * CAP, CAP)])
        pltpu.sync_copy(odb, pdloc.at[pl.ds(b * CAP, CAP)])
        pltpu.sync_copy(cntbuf, pcnt.at[pl.ds(b * 16, 16)])


@functools.partial(
    pl.kernel,
    out_type=jax.ShapeDtypeStruct((NP, H), jnp.float32),
    mesh=_mesh(),
    compiler_params=_sc_params,
    scratch_types=[
        pltpu.VMEM((SUP,), jnp.int32),
        pltpu.VMEM((SUP,), jnp.int32),
        pltpu.SMEM((K,), jnp.int32),
        pltpu.VMEM_SHARED((16, SUP), jnp.int32),
        pltpu.VMEM((K, H), jnp.float32),
        pltpu.VMEM((K, H), jnp.float32),
        pltpu.VMEM((SINK + 1, H), jnp.float32),
        pltpu.VMEM((16,), jnp.int32),
        pltpu.SemaphoreType.DMA,
        pltpu.SemaphoreType.DMA,
    ],
)
def _agg(h_hbm, psrc, pdloc, pcnt, out_hbm, isrc, idloc, sdloc, bdloc, rows0, rows1, acc, cntv, semA, semB):
    w = _wid()
    zero = jnp.zeros((16,), jnp.float32)
    iota = lax.iota(jnp.int32, 16)

    def gather(q16, rbuf, sem):
        return pltpu.async_copy(h_hbm.at[isrc.at[pl.ds(q16, K)]], rbuf, sem)

    def gwait(rbuf, sem):
        pltpu.make_async_copy(h_hbm.at[isrc.at[pl.ds(0, K)]], rbuf, sem).wait()

    def accumulate(q16, rbuf):
        sid = lax.axis_index("s")
        pltpu.sync_copy(bdloc.at[sid, pl.ds(q16, K)], sdloc)

        def edge(e, _):
            ds_row = sdloc[e]
            for f in range(H // 16):
                val = rbuf[e, pl.ds(f * 16, 16)]
                plsc.addupdate(acc.at[ds_row, pl.ds(f * 16, 16)], val)
            return 0

        lax.fori_loop(0, K, edge, 0, unroll=8)

    for j in range(2):
        b = w + NW * j

        def zrow(r, _):
            for f in range(H // 16):
                acc[r, pl.ds(f * 16, 16)] = zero
            return 0

        lax.fori_loop(0, SINK + 1, zrow, 0, unroll=4)

        pltpu.sync_copy(pcnt.at[pl.ds(b * 16, 16)], cntv)
        trips = jnp.max(cntv[...]) // SUP

        def sup_body(ts, _):
            pltpu.sync_copy(psrc.at[pl.ds(b * CAP + ts * SUP, SUP)], isrc)
            pltpu.sync_copy(pdloc.at[pl.ds(b * CAP + ts * SUP, SUP)], idloc)
            pltpu.sync_copy(idloc, bdloc.at[lax.axis_index("s")])
            gather(0, rows0, semA)

            def pair(p, _):
                gather(p * 2 * K + K, rows1, semB)
                gwait(rows0, semA)
                accumulate(p * 2 * K, rows0)

                @pl.when(p < (SUP // (2 * K)) - 1)
                def _():
                    gather(p * 2 * K + 2 * K, rows0, semA)

                gwait(rows1, semB)
                accumulate(p * 2 * K + K, rows1)
                return 0

            lax.fori_loop(0, SUP // (2 * K), pair, 0)
            return 0

        lax.fori_loop(0, trips, sup_body, 0)

        @pl.when(b < NBUCK - 2)
        def _():
            pltpu.sync_copy(acc.at[pl.ds(0, BWN)],
                            out_hbm.at[pl.ds(b * BWN, BWN)])

        @pl.when(b == NBUCK - 2)
        def _():
            pltpu.sync_copy(acc.at[pl.ds(0, N - (NBUCK - 2) * BWN)],
                            out_hbm.at[pl.ds((NBUCK - 2) * BWN,
                                             N - (NBUCK - 2) * BWN)])


def _pre_body(xin, wa, y_ref):
    y_ref[...] = jnp.dot(xin[...], wa[...], preferred_element_type=jnp.float32)


def _pre(xin, wa):
    return pl.pallas_call(
        _pre_body,
        grid=(NG,),
        in_specs=[
            pl.BlockSpec((BR, 16), lambda i: (i, 0)),
            pl.BlockSpec((16, H), lambda i: (0, 0)),
        ],
        out_specs=pl.BlockSpec((BR, H), lambda i: (i, 0)),
        out_shape=jax.ShapeDtypeStruct((NP, H), jnp.float32),
    )(xin, wa)


def _stats_tail(i, u, u_ref, sums_ref):
    rows = i * BR + lax.broadcasted_iota(jnp.int32, (BR, 1), 0)
    u = jnp.where(rows < N, u, 0.0)
    u_ref[...] = u

    @pl.when(i == 0)
    def _():
        sums_ref[...] = jnp.zeros_like(sums_ref)

    sums_ref[0:1, :] += jnp.sum(u, axis=0, keepdims=True)
    sums_ref[1:2, :] += jnp.sum(u * u, axis=0, keepdims=True)


def _ka_body(xin, aggr, wa, ba, u_ref, sums_ref):
    i = pl.program_id(0)
    xa = xin[...] + aggr[...]
    u = jnp.dot(xa, wa[...], preferred_element_type=jnp.float32) + ba[...]
    _stats_tail(i, u, u_ref, sums_ref)


def _ka_add_body(y0, aggr, ba, u_ref, sums_ref):
    i = pl.program_id(0)
    u = y0[...] + aggr[...] + ba[...]
    _stats_tail(i, u, u_ref, sums_ref)


_KA_OUT = [
    jax.ShapeDtypeStruct((NP, H), jnp.float32),
    jax.ShapeDtypeStruct((2, H), jnp.float32),
]
_KA_OUT_SPECS = [
    pl.BlockSpec((BR, H), lambda i: (i, 0)),
    pl.BlockSpec((2, H), lambda i: (0, 0)),
]


def _ka(xin, aggr, wa, ba):
    return pl.pallas_call(
        _ka_body,
        grid=(NG,),
        in_specs=[
            pl.BlockSpec((BR, H), lambda i: (i, 0)),
            pl.BlockSpec((BR, H), lambda i: (i, 0)),
            pl.BlockSpec((H, H), lambda i: (0, 0)),
            pl.BlockSpec((1, H), lambda i: (0, 0)),
        ],
        out_specs=_KA_OUT_SPECS,
        out_shape=_KA_OUT,
    )(xin, aggr, wa, ba.reshape(1, H))


def _ka_add(y0, aggr, ba):
    return pl.pallas_call(
        _ka_add_body,
        grid=(NG,),
        in_specs=[
            pl.BlockSpec((BR, H), lambda i: (i, 0)),
            pl.BlockSpec((BR, H), lambda i: (i, 0)),
            pl.BlockSpec((1, H), lambda i: (0, 0)),
        ],
        out_specs=_KA_OUT_SPECS,
        out_shape=_KA_OUT,
    )(y0, aggr, ba.reshape(1, H))


def _bn_relu(u, sums, g, bt):
    m = sums[0:1, :] * (1.0 / N)
    var = sums[1:2, :] * (1.0 / N) - m * m
    inv = g[...] * lax.rsqrt(var + 1e-5)
    return jnp.maximum(u[...] * inv + (bt[...] - m * inv), 0.0)


def _kb_body(u, sums, g, bt, wb, bb, h_ref):
    t = _bn_relu(u, sums, g, bt)
    h_ref[...] = jnp.maximum(
        jnp.dot(t, wb[...], preferred_element_type=jnp.float32) + bb[...], 0.0)


def _kb3_body(u, sums, g, bt, wb, bb, out_ref):
    i = pl.program_id(0)
    t = _bn_relu(u, sums, g, bt)
    h = jnp.maximum(
        jnp.dot(t, wb[...], preferred_element_type=jnp.float32) + bb[...], 0.0)
    rows = i * BR + lax.broadcasted_iota(jnp.int32, (BR, 1), 0)
    h = jnp.where(rows < N, h, 0.0)

    @pl.when(i == 0)
    def _():
        out_ref[...] = jnp.zeros_like(out_ref)

    out_ref[...] += jnp.sum(h, axis=0, keepdims=True)


def _kb(u, sums, g, bt, wb, bb, last):
    in_specs = [
        pl.BlockSpec((BR, H), lambda i: (i, 0)),
        pl.BlockSpec((2, H), lambda i: (0, 0)),
        pl.BlockSpec((1, H), lambda i: (0, 0)),
        pl.BlockSpec((1, H), lambda i: (0, 0)),
        pl.BlockSpec((H, H), lambda i: (0, 0)),
        pl.BlockSpec((1, H), lambda i: (0, 0)),
    ]
    if last:
        out_spec = pl.BlockSpec((1, H), lambda i: (0, 0))
        out_shape = jax.ShapeDtypeStruct((1, H), jnp.float32)
        body = _kb3_body
    else:
        out_spec = pl.BlockSpec((BR, H), lambda i: (i, 0))
        out_shape = jax.ShapeDtypeStruct((NP, H), jnp.float32)
        body = _kb_body
    return pl.pallas_call(
        body, grid=(NG,), in_specs=in_specs, out_specs=out_spec,
        out_shape=out_shape,
    )(u, sums, g.reshape(1, H), bt.reshape(1, H), wb, bb.reshape(1, H))


def kernel(x, edge_index,
           W1_1, b1_1, g1, bt1, W1_2, b1_2,
           W2_1, b2_1, g2, bt2, W2_2, b2_2,
           W3_1, b3_1, g3, bt3, W3_2, b3_2):
    src = edge_index[0]
    dst = edge_index[1]

    x_p = jnp.zeros((NP, 16), jnp.float32).at[:N, :9].set(x)
    w1_p = jnp.zeros((16, H), jnp.float32).at[:9, :].set(W1_1)

    psrc, pdloc, pcnt = _partition(src, dst)

    y0 = _pre(x_p, w1_p)
    agg1 = _agg(y0, psrc, pdloc, pcnt)
    u1, s1 = _ka_add(y0, agg1, b1_1)
    h1 = _kb(u1, s1, g1, bt1, W1_2, b1_2, last=False)

    agg2 = _agg(h1, psrc, pdloc, pcnt)
    u2, s2 = _ka(h1, agg2, W2_1, b2_1)
    h2 = _kb(u2, s2, g2, bt2, W2_2, b2_2, last=False)

    agg3 = _agg(h2, psrc, pdloc, pcnt)
    u3, s3 = _ka(h2, agg3, W3_1, b3_1)
    out = _kb(u3, s3, g3, bt3, W3_2, b3_2, last=True)

    return out.reshape(H)


# speedup vs baseline: 2.7590x; 1.0149x over previous
import functools

import jax
import jax.numpy as jnp
from jax import lax
from jax.experimental import pallas as pl
from jax.experimental.pallas import tpu as pltpu
from jax.experimental.pallas import tpu_sc as plsc

N = 50000
E = 800000
H = 128
NP = 50176
BR = 512
NG = NP // BR

NW = 32
NBUCK = 64
BWN = 800
CAP = 20000
ECH = 16000
K = 64
SUP = 1024
SINK = 800
ZR = 89

_mesh = lambda: plsc.VectorSubcoreMesh(core_axis_name="c", subcore_axis_name="s")
_sc_params = pltpu.CompilerParams(needs_layout_passes=False)


def _wid():
    return lax.axis_index("s") * 2 + lax.axis_index("c")


@functools.partial(
    pl.kernel,
    out_type=(
        jax.ShapeDtypeStruct((NBUCK * CAP,), jnp.int32),
        jax.ShapeDtypeStruct((NBUCK * CAP,), jnp.int32),
        jax.ShapeDtypeStruct((NBUCK * 16,), jnp.int32),
    ),
    mesh=_mesh(),
    compiler_params=_sc_params,
    scratch_types=[
        pltpu.VMEM((ECH,), jnp.int32),
        pltpu.VMEM((ECH,), jnp.int32),
        pltpu.VMEM((CAP,), jnp.int32),
        pltpu.VMEM((CAP,), jnp.int32),
        pltpu.VMEM((CAP,), jnp.int32),
        pltpu.VMEM((CAP,), jnp.int32),
        pltpu.VMEM((16,), jnp.int32),
    ],
)
def _partition(src_hbm, dst_hbm, psrc, pdloc, pcnt,
               sbuf, dbuf, os0, od0, os1, od1, cntbuf):
    w = _wid()
    lo0 = w * BWN
    lo1 = (w + NW) * BWN

    def chunk(c, carry):
        o0, o1 = carry
        pltpu.sync_copy(src_hbm.at[pl.ds(c * ECH, ECH)], sbuf)
        pltpu.sync_copy(dst_hbm.at[pl.ds(c * ECH, ECH)], dbuf)

        def vec(i, carry):
            o0, o1 = carry
            d = dbuf[pl.ds(i * 16, 16)]
            s = sbuf[pl.ds(i * 16, 16)]
            m0 = (d >= lo0) & (d < lo0 + BWN)
            m1 = (d >= lo1) & (d < lo1 + BWN)
            n0 = plsc.all_reduce_population_count(m0)
            n1 = plsc.all_reduce_population_count(m1)
            pc0 = plsc.cumsum(m0.astype(jnp.int32))
            pc1 = plsc.cumsum(m1.astype(jnp.int32))
            i0 = o0 + pc0 - 1
            i1 = o1 + pc1 - 1
            plsc.store_scatter(od0, [i0], d - lo0, mask=m0)
            plsc.store_scatter(os0, [i0], s, mask=m0)
            plsc.store_scatter(od1, [i1], d - lo1, mask=m1)
            plsc.store_scatter(os1, [i1], s, mask=m1)
            return o0 + n0, o1 + n1

        return lax.fori_loop(0, ECH // 16, vec, (o0, o1), unroll=8)

    zv = jnp.zeros((16,), jnp.int32)
    o0v, o1v = lax.fori_loop(0, E // ECH, chunk, (zv, zv))
    o0 = o0v[0]
    o1 = o1v[0]

    sink_d = jnp.full((16,), SINK, jnp.int32)
    iota16 = lax.iota(jnp.int32, 16)
    for j, (osb, odb, o) in enumerate(((os0, od0, o0), (os1, od1, o1))):
        b = w + NW * j
        for t in range(SUP // 16):
            odb[pl.ds(o + t * 16, 16)] = sink_d
            osb[pl.ds(o + t * 16, 16)] = w * 1536 + t * 16 + iota16
        cnt_pad = ((o + SUP - 1) // SUP) * SUP
        cntbuf[...] = jnp.full((16,), 0, jnp.int32) + cnt_pad
        pltpu.sync_copy(osb, psrc.at[pl.ds(b * CAP, CAP)])
        pltpu.sync_copy(odb, pdloc.at[pl.ds(b * CAP, CAP)])
        pltpu.sync_copy(cntbuf, pcnt.at[pl.ds(b * 16, 16)])


@functools.partial(
    pl.kernel,
    out_type=jax.ShapeDtypeStruct((NP, H), jnp.float32),
    mesh=_mesh(),
    compiler_params=_sc_params,
    scratch_types=[
        pltpu.VMEM((SUP,), jnp.int32),
        pltpu.VMEM((SUP,), jnp.int32),
        pltpu.SMEM((2 * K,), jnp.int32),
        pltpu.VMEM_SHARED((16, SUP), jnp.int32),
        pltpu.VMEM((K, H), jnp.float32),
        pltpu.VMEM((K, H), jnp.float32),
        pltpu.VMEM((SINK + 1, H), jnp.float32),
        pltpu.VMEM((16,), jnp.int32),
        pltpu.SemaphoreType.DMA,
        pltpu.SemaphoreType.DMA,
    ],
)
def _agg(h_hbm, psrc, pdloc, pcnt, out_hbm, isrc, idloc, sdloc, bdloc, rows0, rows1, acc, cntv, semA, semB):
    w = _wid()
    zero = jnp.zeros((16,), jnp.float32)
    iota = lax.iota(jnp.int32, 16)

    def gather(q16, rbuf, sem):
        return pltpu.async_copy(h_hbm.at[isrc.at[pl.ds(q16, K)]], rbuf, sem)

    def gwait(rbuf, sem):
        pltpu.make_async_copy(h_hbm.at[isrc.at[pl.ds(0, K)]], rbuf, sem).wait()

    def accumulate(soff, rbuf):
        def edge(e, _):
            ds_row = sdloc[soff + e]
            for f in range(H // 16):
                val = rbuf[e, pl.ds(f * 16, 16)]
                plsc.addupdate(acc.at[ds_row, pl.ds(f * 16, 16)], val)
            return 0

        lax.fori_loop(0, K, edge, 0, unroll=8)

    for j in range(2):
        b = w + NW * j

        def zrow(r, _):
            for f in range(H // 16):
                acc[r, pl.ds(f * 16, 16)] = zero
            return 0

        lax.fori_loop(0, SINK + 1, zrow, 0, unroll=4)

        pltpu.sync_copy(pcnt.at[pl.ds(b * 16, 16)], cntv)
        trips = jnp.max(cntv[...]) // SUP

        def sup_body(ts, _):
            pltpu.sync_copy(psrc.at[pl.ds(b * CAP + ts * SUP, SUP)], isrc)
            pltpu.sync_copy(pdloc.at[pl.ds(b * CAP + ts * SUP, SUP)], idloc)
            pltpu.sync_copy(idloc, bdloc.at[lax.axis_index("s")])
            gather(0, rows0, semA)

            def pair(p, _):
                gather(p * 2 * K + K, rows1, semB)
                pltpu.sync_copy(bdloc.at[lax.axis_index("s"),
                                         pl.ds(p * 2 * K, 2 * K)], sdloc)
                gwait(rows0, semA)
                accumulate(0, rows0)

                @pl.when(p < (SUP // (2 * K)) - 1)
                def _():
                    gather(p * 2 * K + 2 * K, rows0, semA)

                gwait(rows1, semB)
                accumulate(K, rows1)
                return 0

            lax.fori_loop(0, SUP // (2 * K), pair, 0)
            return 0

        lax.fori_loop(0, trips, sup_body, 0)

        @pl.when(b < NBUCK - 2)
        def _():
            pltpu.sync_copy(acc.at[pl.ds(0, BWN)],
                            out_hbm.at[pl.ds(b * BWN, BWN)])

        @pl.when(b == NBUCK - 2)
        def _():
            pltpu.sync_copy(acc.at[pl.ds(0, N - (NBUCK - 2) * BWN)],
                            out_hbm.at[pl.ds((NBUCK - 2) * BWN,
                                             N - (NBUCK - 2) * BWN)])


def _pre_body(xin, wa, y_ref):
    y_ref[...] = jnp.dot(xin[...], wa[...], preferred_element_type=jnp.float32)


def _pre(xin, wa):
    return pl.pallas_call(
        _pre_body,
        grid=(NG,),
        in_specs=[
            pl.BlockSpec((BR, 16), lambda i: (i, 0)),
            pl.BlockSpec((16, H), lambda i: (0, 0)),
        ],
        out_specs=pl.BlockSpec((BR, H), lambda i: (i, 0)),
        out_shape=jax.ShapeDtypeStruct((NP, H), jnp.float32),
    )(xin, wa)


def _stats_tail(i, u, u_ref, sums_ref):
    rows = i * BR + lax.broadcasted_iota(jnp.int32, (BR, 1), 0)
    u = jnp.where(rows < N, u, 0.0)
    u_ref[...] = u

    @pl.when(i == 0)
    def _():
        sums_ref[...] = jnp.zeros_like(sums_ref)

    sums_ref[0:1, :] += jnp.sum(u, axis=0, keepdims=True)
    sums_ref[1:2, :] += jnp.sum(u * u, axis=0, keepdims=True)


def _ka_body(xin, aggr, wa, ba, u_ref, sums_ref):
    i = pl.program_id(0)
    xa = xin[...] + aggr[...]
    u = jnp.dot(xa, wa[...], preferred_element_type=jnp.float32) + ba[...]
    _stats_tail(i, u, u_ref, sums_ref)


def _ka_add_body(y0, aggr, ba, u_ref, sums_ref):
    i = pl.program_id(0)
    u = y0[...] + aggr[...] + ba[...]
    _stats_tail(i, u, u_ref, sums_ref)


_KA_OUT = [
    jax.ShapeDtypeStruct((NP, H), jnp.float32),
    jax.ShapeDtypeStruct((2, H), jnp.float32),
]
_KA_OUT_SPECS = [
    pl.BlockSpec((BR, H), lambda i: (i, 0)),
    pl.BlockSpec((2, H), lambda i: (0, 0)),
]


def _ka(xin, aggr, wa, ba):
    return pl.pallas_call(
        _ka_body,
        grid=(NG,),
        in_specs=[
            pl.BlockSpec((BR, H), lambda i: (i, 0)),
            pl.BlockSpec((BR, H), lambda i: (i, 0)),
            pl.BlockSpec((H, H), lambda i: (0, 0)),
            pl.BlockSpec((1, H), lambda i: (0, 0)),
        ],
        out_specs=_KA_OUT_SPECS,
        out_shape=_KA_OUT,
    )(xin, aggr, wa, ba.reshape(1, H))


def _ka_add(y0, aggr, ba):
    return pl.pallas_call(
        _ka_add_body,
        grid=(NG,),
        in_specs=[
            pl.BlockSpec((BR, H), lambda i: (i, 0)),
            pl.BlockSpec((BR, H), lambda i: (i, 0)),
            pl.BlockSpec((1, H), lambda i: (0, 0)),
        ],
        out_specs=_KA_OUT_SPECS,
        out_shape=_KA_OUT,
    )(y0, aggr, ba.reshape(1, H))


def _bn_relu(u, sums, g, bt):
    m = sums[0:1, :] * (1.0 / N)
    var = sums[1:2, :] * (1.0 / N) - m * m
    inv = g[...] * lax.rsqrt(var + 1e-5)
    return jnp.maximum(u[...] * inv + (bt[...] - m * inv), 0.0)


def _kb_body(u, sums, g, bt, wb, bb, h_ref):
    t = _bn_relu(u, sums, g, bt)
    h_ref[...] = jnp.maximum(
        jnp.dot(t, wb[...], preferred_element_type=jnp.float32) + bb[...], 0.0)


def _kb3_body(u, sums, g, bt, wb, bb, out_ref):
    i = pl.program_id(0)
    t = _bn_relu(u, sums, g, bt)
    h = jnp.maximum(
        jnp.dot(t, wb[...], preferred_element_type=jnp.float32) + bb[...], 0.0)
    rows = i * BR + lax.broadcasted_iota(jnp.int32, (BR, 1), 0)
    h = jnp.where(rows < N, h, 0.0)

    @pl.when(i == 0)
    def _():
        out_ref[...] = jnp.zeros_like(out_ref)

    out_ref[...] += jnp.sum(h, axis=0, keepdims=True)


def _kb(u, sums, g, bt, wb, bb, last):
    in_specs = [
        pl.BlockSpec((BR, H), lambda i: (i, 0)),
        pl.BlockSpec((2, H), lambda i: (0, 0)),
        pl.BlockSpec((1, H), lambda i: (0, 0)),
        pl.BlockSpec((1, H), lambda i: (0, 0)),
        pl.BlockSpec((H, H), lambda i: (0, 0)),
        pl.BlockSpec((1, H), lambda i: (0, 0)),
    ]
    if last:
        out_spec = pl.BlockSpec((1, H), lambda i: (0, 0))
        out_shape = jax.ShapeDtypeStruct((1, H), jnp.float32)
        body = _kb3_body
    else:
        out_spec = pl.BlockSpec((BR, H), lambda i: (i, 0))
        out_shape = jax.ShapeDtypeStruct((NP, H), jnp.float32)
        body = _kb_body
    return pl.pallas_call(
        body, grid=(NG,), in_specs=in_specs, out_specs=out_spec,
        out_shape=out_shape,
    )(u, sums, g.reshape(1, H), bt.reshape(1, H), wb, bb.reshape(1, H))


def kernel(x, edge_index,
           W1_1, b1_1, g1, bt1, W1_2, b1_2,
           W2_1, b2_1, g2, bt2, W2_2, b2_2,
           W3_1, b3_1, g3, bt3, W3_2, b3_2):
    src = edge_index[0]
    dst = edge_index[1]

    x_p = jnp.zeros((NP, 16), jnp.float32).at[:N, :9].set(x)
    w1_p = jnp.zeros((16, H), jnp.float32).at[:9, :].set(W1_1)

    psrc, pdloc, pcnt = _partition(src, dst)

    y0 = _pre(x_p, w1_p)
    agg1 = _agg(y0, psrc, pdloc, pcnt)
    u1, s1 = _ka_add(y0, agg1, b1_1)
    h1 = _kb(u1, s1, g1, bt1, W1_2, b1_2, last=False)

    agg2 = _agg(h1, psrc, pdloc, pcnt)
    u2, s2 = _ka(h1, agg2, W2_1, b2_1)
    h2 = _kb(u2, s2, g2, bt2, W2_2, b2_2, last=False)

    agg3 = _agg(h2, psrc, pdloc, pcnt)
    u3, s3 = _ka(h2, agg3, W3_1, b3_1)
    out = _kb(u3, s3, g3, bt3, W3_2, b3_2, last=True)

    return out.reshape(H)
